# Initial kernel scaffold; baseline (speedup 1.0000x reference)
import jax
import jax.numpy as jnp
from jax.experimental import pallas as pl


def kernel(state):
    raise NotImplementedError("write your pallas kernel here")


# speedup vs baseline: 1.3523x; 1.3523x over previous
import jax
import jax.numpy as jnp
from jax.experimental import pallas as pl

_WIDTH = 1000
_FEATURE_DIM = 100000

_RB = 256
_CB = 8192


def _onehot_block(state_ref, out_ref):
    j = pl.program_id(1)
    idx = state_ref[:, 0:1] + _WIDTH * state_ref[:, 1:2]
    local = idx - j * _CB
    cols = jax.lax.broadcasted_iota(jnp.int32, (_RB, _CB), 1)
    out_ref[...] = (cols == local).astype(jnp.float32)


def kernel(state):
    n = state.shape[0]
    grid = (n // _RB, pl.cdiv(_FEATURE_DIM, _CB))
    return pl.pallas_call(
        _onehot_block,
        grid=grid,
        in_specs=[pl.BlockSpec((_RB, 2), lambda i, j: (i, 0))],
        out_specs=pl.BlockSpec((_RB, _CB), lambda i, j: (i, j)),
        out_shape=jax.ShapeDtypeStruct((n, _FEATURE_DIM), jnp.float32),
    )(state)
